# Initial kernel scaffold; baseline (speedup 1.0000x reference)
import jax
import jax.numpy as jnp
from jax.experimental import pallas as pl


def kernel(x, edge_index, c, edge_attr, batch, We0, be0, W10, b10, W20, b20, We1, be1, W11, b11, W21, b21, We2, be2, W12, b12, W22, b22, Wp, bp):
    raise NotImplementedError("write your pallas kernel here")


# speedup vs baseline: 1.6764x; 1.6764x over previous
import functools

import jax
import jax.numpy as jnp
from jax import lax
from jax.experimental import pallas as pl
from jax.experimental.pallas import tpu as pltpu
from jax.experimental.pallas import tpu_sc as plsc

F32 = jnp.float32

_N = 10000
_N2 = 10112
_E = 320000
_EPAD = 327680
_C = 128
_NS = 16
_EB = 2048
_NB = 1000


def _edge_stage(table, src, dst, t, *, n_nodes, dh, e_pad, edge_split):
    esc = e_pad // 2 if edge_split else e_pad
    per_tile = esc // _NS
    chunks = per_tile // _C
    rpt = _N2 // _NS
    mesh = plsc.VectorSubcoreMesh(core_axis_name="c", subcore_axis_name="s")

    @functools.partial(
        pl.kernel,
        out_type=jax.ShapeDtypeStruct((2 * _N2, dh), F32),
        mesh=mesh,
        scratch_types=[
            pltpu.VMEM((_C,), jnp.int32),
            pltpu.VMEM((_C,), jnp.int32),
            pltpu.VMEM((_C, dh), F32),
            pltpu.VMEM((_C, dh), F32),
            pltpu.VMEM_SHARED((_N2, dh), F32),
            pltpu.SemaphoreType.DMA,
        ],
    )
    def k(table_hbm, src_hbm, dst_hbm, t_hbm, out_hbm,
          sidx_v, didx_v, rows_v, t_v, slab_sh, sem):
        cid = lax.axis_index("c")
        sid = lax.axis_index("s")
        cks = [(o, min(_C, rpt - o)) for o in range(0, rpt, _C)]

        def zrow(r, _):
            for j in range(dh // 16):
                rows_v[r, pl.ds(j * 16, 16)] = jnp.zeros((16,), F32)
            return 0
        lax.fori_loop(0, _C, zrow, 0)
        for o, w in cks:
            pltpu.sync_copy(rows_v.at[pl.ds(0, w)],
                            slab_sh.at[pl.ds(sid * rpt + o, w)])
        plsc.subcore_barrier()

        sc_e0 = cid * esc if edge_split else 0
        t_base = cid * esc if edge_split else cid * e_pad

        def body(ck, _):
            be = sid * per_tile + ck * _C
            pltpu.sync_copy(src_hbm.at[pl.ds(sc_e0 + be, _C)], sidx_v)
            pltpu.sync_copy(dst_hbm.at[pl.ds(sc_e0 + be, _C)], didx_v)
            if not edge_split:
                for j in range(_C // 16):
                    sl = pl.ds(j * 16, 16)
                    sidx_v[sl] = sidx_v[sl] + cid * _N2
            pltpu.async_copy(table_hbm.at[sidx_v], rows_v, sem).wait()
            pltpu.sync_copy(t_hbm.at[pl.ds(t_base + be, _C)], t_v)

            def crow(r, _):
                for j in range(dh // 16):
                    sl = pl.ds(j * 16, 16)
                    rows_v[r, sl] = jnp.maximum(rows_v[r, sl] + t_v[r, sl],
                                                jnp.zeros((16,), F32))
                return 0
            lax.fori_loop(0, _C, crow, 0)
            pltpu.sync_copy(rows_v, slab_sh.at[didx_v], add=True)
            return 0
        lax.fori_loop(0, chunks, body, 0)
        plsc.subcore_barrier()

        for o, w in cks:
            r0 = sid * rpt + o
            pltpu.sync_copy(slab_sh.at[pl.ds(r0, w)], rows_v.at[pl.ds(0, w)])
            pltpu.sync_copy(rows_v.at[pl.ds(0, w)],
                            out_hbm.at[pl.ds(cid * _N2 + r0, w)])

    return k(table, src, dst, t)


def _edge_transform(eap, We0, be0, We1, be1, We2, be2):
    grid = _EPAD // _EB

    def body(ea_ref, W0_ref, b0_ref, W1_ref, b1_ref, W2_ref, b2_ref,
             t0_ref, t1_ref, t2_ref):
        ea = ea_ref[...]
        a0 = jnp.dot(ea, W0_ref[...], preferred_element_type=F32) + b0_ref[...]
        t0_ref[...] = a0
        a1 = jnp.dot(ea, W1_ref[...], preferred_element_type=F32) + b1_ref[...]
        t1_ref[0] = a1[:, :128]
        t1_ref[1] = a1[:, 128:]
        a2 = jnp.dot(ea, W2_ref[...], preferred_element_type=F32) + b2_ref[...]
        t2_ref[0] = a2[:, :128]
        t2_ref[1] = a2[:, 128:]

    t0, t1, t2 = pl.pallas_call(
        body,
        grid=(grid,),
        in_specs=[
            pl.BlockSpec((_EB, 16), lambda i: (i, 0)),
            pl.BlockSpec((16, 128), lambda i: (0, 0)),
            pl.BlockSpec((1, 128), lambda i: (0, 0)),
            pl.BlockSpec((16, 256), lambda i: (0, 0)),
            pl.BlockSpec((1, 256), lambda i: (0, 0)),
            pl.BlockSpec((16, 256), lambda i: (0, 0)),
            pl.BlockSpec((1, 256), lambda i: (0, 0)),
        ],
        out_specs=[
            pl.BlockSpec((_EB, 128), lambda i: (i, 0)),
            pl.BlockSpec((2, _EB, 128), lambda i: (0, i, 0)),
            pl.BlockSpec((2, _EB, 128), lambda i: (0, i, 0)),
        ],
        out_shape=[
            jax.ShapeDtypeStruct((_EPAD, 128), F32),
            jax.ShapeDtypeStruct((2, _EPAD, 128), F32),
            jax.ShapeDtypeStruct((2, _EPAD, 128), F32),
        ],
    )(eap, We0, be0.reshape(1, -1), We1, be1.reshape(1, -1),
      We2, be2.reshape(1, -1))
    return t0, t1.reshape(2 * _EPAD, 128), t2.reshape(2 * _EPAD, 128)


def _mlp(h, a, cc, W1, b1, W2, b2, *, first, last_relu):
    grid = _N // _NB

    def body(h_ref, a_ref, c_ref, W1_ref, b1_ref, W2_ref, b2_ref, o_ref):
        if first:
            z = h_ref[...] + a_ref[0] + a_ref[1]
        else:
            z = jnp.concatenate([h_ref[0] + a_ref[0],
                                 h_ref[1] + a_ref[1]], axis=1)
        z = jnp.concatenate([z, c_ref[...]], axis=1)
        hmid = jnp.maximum(
            jnp.dot(z, W1_ref[...], preferred_element_type=F32) + b1_ref[...],
            0.0)
        hnew = jnp.dot(hmid, W2_ref[...], preferred_element_type=F32) \
            + b2_ref[...]
        if last_relu:
            hnew = jnp.maximum(hnew, 0.0)
        o_ref[0] = hnew[:, :128]
        o_ref[1] = hnew[:, 128:]

    din = 128 if first else 256
    h_spec = (pl.BlockSpec((_NB, 128), lambda i: (i, 0)) if first
              else pl.BlockSpec((2, _NB, 128), lambda i: (0, i, 0)))
    out = pl.pallas_call(
        body,
        grid=(grid,),
        in_specs=[
            h_spec,
            pl.BlockSpec((2, _NB, 128), lambda i: (0, i, 0)),
            pl.BlockSpec((_NB, 16), lambda i: (i, 0)),
            pl.BlockSpec((din + 16, 512), lambda i: (0, 0)),
            pl.BlockSpec((1, 512), lambda i: (0, 0)),
            pl.BlockSpec((512, 256), lambda i: (0, 0)),
            pl.BlockSpec((1, 256), lambda i: (0, 0)),
        ],
        out_specs=pl.BlockSpec((2, _NB, 128), lambda i: (0, i, 0)),
        out_shape=jax.ShapeDtypeStruct((2, _N2, 128), F32),
    )(h, a, cc, W1, b1.reshape(1, -1), W2, b2.reshape(1, -1))
    return out


def _readout(x, h1, h2, h3, batch3, Wp, bp, n_graphs):
    grid = _N // _NB

    def body(x_ref, h1_ref, h2_ref, h3_ref, b_ref, Wp_ref, bp_ref, o_ref,
             s_ref):
        i = pl.program_id(0)

        @pl.when(i == 0)
        def _():
            s_ref[...] = jnp.zeros_like(s_ref)

        b = b_ref[0, 0, :]
        iota = lax.broadcasted_iota(jnp.int32, (n_graphs, _NB), 0)
        onehot = (b[None, :] == iota).astype(F32)
        hcat = jnp.concatenate(
            [x_ref[...], h1_ref[0], h1_ref[1],
             h2_ref[0], h2_ref[1], h3_ref[0], h3_ref[1]], axis=1)
        s_ref[...] += jnp.dot(onehot, hcat, preferred_element_type=F32)

        @pl.when(i == grid - 1)
        def _():
            o_ref[...] = jnp.dot(s_ref[...], Wp_ref[...],
                                 preferred_element_type=F32) + bp_ref[...]

    return pl.pallas_call(
        body,
        grid=(grid,),
        in_specs=[
            pl.BlockSpec((_NB, 128), lambda i: (i, 0)),
            pl.BlockSpec((2, _NB, 128), lambda i: (0, i, 0)),
            pl.BlockSpec((2, _NB, 128), lambda i: (0, i, 0)),
            pl.BlockSpec((2, _NB, 128), lambda i: (0, i, 0)),
            pl.BlockSpec((1, 1, _NB), lambda i: (i, 0, 0)),
            pl.BlockSpec((896, 10), lambda i: (0, 0)),
            pl.BlockSpec((1, 10), lambda i: (0, 0)),
        ],
        out_specs=pl.BlockSpec((n_graphs, 10), lambda i: (0, 0)),
        out_shape=jax.ShapeDtypeStruct((n_graphs, 10), F32),
        scratch_shapes=[pltpu.VMEM((n_graphs, 896), F32)],
    )(x, h1, h2, h3, batch3, Wp, bp.reshape(1, -1))


def kernel(x, edge_index, c, edge_attr, batch,
           We0, be0, W10, b10, W20, b20,
           We1, be1, W11, b11, W21, b21,
           We2, be2, W12, b12, W22, b22,
           Wp, bp):
    pad = _EPAD - _E
    src = jnp.concatenate([edge_index[0], jnp.zeros((pad,), jnp.int32)])
    dst = jnp.concatenate([edge_index[1], jnp.full((pad,), _N, jnp.int32)])
    eap = jnp.concatenate([edge_attr, jnp.zeros((pad, 16), F32)])

    t0, t1, t2 = _edge_transform(eap, We0, be0, We1, be1, We2, be2)

    a0 = _edge_stage(x, src, dst, t0,
                     n_nodes=_N, dh=128, e_pad=_EPAD, edge_split=True)
    h1 = _mlp(x, a0.reshape(2, _N2, 128), c, W10, b10, W20, b20,
              first=True, last_relu=True)

    a1 = _edge_stage(h1.reshape(2 * _N2, 128), src, dst, t1,
                     n_nodes=_N, dh=128, e_pad=_EPAD, edge_split=False)
    h2 = _mlp(h1, a1.reshape(2, _N2, 128), c, W11, b11, W21, b21,
              first=False, last_relu=True)

    a2 = _edge_stage(h2.reshape(2 * _N2, 128), src, dst, t2,
                     n_nodes=_N, dh=128, e_pad=_EPAD, edge_split=False)
    h3 = _mlp(h2, a2.reshape(2, _N2, 128), c, W12, b12, W22, b22,
              first=False, last_relu=False)

    batch3 = batch.reshape(_N // _NB, 1, _NB)
    return _readout(x, h1, h2, h3, batch3, Wp, bp, 64)


# speedup vs baseline: 2.4637x; 1.4697x over previous
import functools

import jax
import jax.numpy as jnp
from jax import lax
from jax.experimental import pallas as pl
from jax.experimental.pallas import tpu as pltpu
from jax.experimental.pallas import tpu_sc as plsc

F32 = jnp.float32

_N = 10000
_N2 = 10112
_E = 320000
_EPAD = 327680
_C = 80
_NS = 16
_EB = 2048
_NB = 1000


def _edge_stage(table, j, t, *, dh, e_pad, edge_split):
    esc = e_pad // 2 if edge_split else e_pad
    per_tile = esc // _NS
    chunks = per_tile // _C
    nch = esc // _C
    rpt = _N2 // _NS
    mesh = plsc.VectorSubcoreMesh(core_axis_name="c", subcore_axis_name="s")

    @functools.partial(
        pl.kernel,
        out_type=jax.ShapeDtypeStruct((2 * _N2, dh), F32),
        mesh=mesh,
        scratch_types=[
            pltpu.VMEM((4, 2, _C), jnp.int32),
            pltpu.VMEM((2, _C, dh), F32),
            pltpu.VMEM((2, _C, dh), F32),
            pltpu.VMEM_SHARED((_N2, dh), F32),
            pltpu.SemaphoreType.DMA((4,)),
            pltpu.SemaphoreType.DMA((2,)),
            pltpu.SemaphoreType.DMA((2,)),
            pltpu.SemaphoreType.DMA((2,)),
        ],
    )
    def k(table_hbm, j_hbm, t_hbm, out_hbm,
          idx, rows, tb, slab_sh, isem, gsem, tsem, ssem):
        cid = lax.axis_index("c")
        sid = lax.axis_index("s")
        cks = [(o, min(_C, rpt - o)) for o in range(0, rpt, _C)]

        def zrow(r, _):
            for jj in range(dh // 16):
                rows[0, r, pl.ds(jj * 16, 16)] = jnp.zeros((16,), F32)
            return 0
        lax.fori_loop(0, _C, zrow, 0)
        for o, w in cks:
            pltpu.sync_copy(rows.at[0, pl.ds(0, w)],
                            slab_sh.at[pl.ds(sid * rpt + o, w)])
        plsc.subcore_barrier()

        jbase = cid * nch + sid * chunks
        t_base = (cid * esc if edge_split else cid * e_pad) + sid * per_tile

        def issue_idx(ck, si):
            pltpu.async_copy(j_hbm.at[jbase + ck], idx.at[si], isem.at[si])

        def wait_idx(ck, si):
            pltpu.make_async_copy(j_hbm.at[jbase + ck], idx.at[si],
                                  isem.at[si]).wait()

        def issue_gt(ck, s, si):
            pltpu.async_copy(table_hbm.at[idx.at[si, 0]], rows.at[s],
                             gsem.at[s])
            pltpu.async_copy(t_hbm.at[pl.ds(t_base + ck * _C, _C)], tb.at[s],
                             tsem.at[s])

        def wait_gt(ck, s, si):
            pltpu.make_async_copy(table_hbm.at[idx.at[si, 0]], rows.at[s],
                                  gsem.at[s]).wait()
            pltpu.make_async_copy(t_hbm.at[pl.ds(t_base + ck * _C, _C)],
                                  tb.at[s], tsem.at[s]).wait()

        def compute(s):
            def crow(r, _):
                for jj in range(dh // 16):
                    sl = pl.ds(jj * 16, 16)
                    rows[s, r, sl] = jnp.maximum(
                        rows[s, r, sl] + tb[s, r, sl], jnp.zeros((16,), F32))
                return 0
            lax.fori_loop(0, _C, crow, 0)

        def issue_scatter(s, si):
            pltpu.async_copy(rows.at[s], slab_sh.at[idx.at[si, 1]],
                             ssem.at[s], add=True)

        def wait_scatter(s, si):
            pltpu.make_async_copy(rows.at[s], slab_sh.at[idx.at[si, 1]],
                                  ssem.at[s]).wait()

        issue_idx(0, 0)
        issue_idx(1, 1)

        def pair4(p, _):
            for s4 in range(4):
                ck = 4 * p + s4
                s = s4 % 2
                wait_idx(ck, s4)

                @pl.when(ck >= 2)
                def _():
                    wait_scatter(s, (s4 + 2) % 4)
                issue_gt(ck, s, s4)

                @pl.when(ck + 2 < chunks)
                def _():
                    issue_idx(ck + 2, (s4 + 2) % 4)

                @pl.when(ck >= 1)
                def _():
                    wait_gt(ck - 1, 1 - s, (s4 + 3) % 4)
                    compute(1 - s)
                    issue_scatter(1 - s, (s4 + 3) % 4)
            return 0
        lax.fori_loop(0, chunks // 4, pair4, 0)

        lc = chunks - 1
        wait_gt(lc, 1, 3)
        compute(1)
        issue_scatter(1, 3)
        wait_scatter(0, 2)
        wait_scatter(1, 3)
        plsc.subcore_barrier()

        for o, w in cks:
            r0 = sid * rpt + o
            pltpu.sync_copy(slab_sh.at[pl.ds(r0, w)], rows.at[0, pl.ds(0, w)])
            pltpu.sync_copy(rows.at[0, pl.ds(0, w)],
                            out_hbm.at[pl.ds(cid * _N2 + r0, w)])

    return k(table, j, t)


def _mk_chunks(src, dst, *, e_pad, edge_split):
    if edge_split:
        esc = e_pad // 2
        parts = [(src[:esc], dst[:esc]), (src[esc:], dst[esc:])]
    else:
        parts = [(src, dst), (src + _N2, dst)]
    return jnp.concatenate([
        jnp.stack([s.reshape(-1, _C), d.reshape(-1, _C)], axis=1)
        for s, d in parts], axis=0)


def _edge_transform(eap, We0, be0, We1, be1, We2, be2):
    grid = _EPAD // _EB

    def body(ea_ref, W0_ref, b0_ref, W1_ref, b1_ref, W2_ref, b2_ref,
             t0_ref, t1_ref, t2_ref):
        ea = ea_ref[...]
        a0 = jnp.dot(ea, W0_ref[...], preferred_element_type=F32) + b0_ref[...]
        t0_ref[...] = a0
        a1 = jnp.dot(ea, W1_ref[...], preferred_element_type=F32) + b1_ref[...]
        t1_ref[0] = a1[:, :128]
        t1_ref[1] = a1[:, 128:]
        a2 = jnp.dot(ea, W2_ref[...], preferred_element_type=F32) + b2_ref[...]
        t2_ref[0] = a2[:, :128]
        t2_ref[1] = a2[:, 128:]

    t0, t1, t2 = pl.pallas_call(
        body,
        grid=(grid,),
        in_specs=[
            pl.BlockSpec((_EB, 16), lambda i: (i, 0)),
            pl.BlockSpec((16, 128), lambda i: (0, 0)),
            pl.BlockSpec((1, 128), lambda i: (0, 0)),
            pl.BlockSpec((16, 256), lambda i: (0, 0)),
            pl.BlockSpec((1, 256), lambda i: (0, 0)),
            pl.BlockSpec((16, 256), lambda i: (0, 0)),
            pl.BlockSpec((1, 256), lambda i: (0, 0)),
        ],
        out_specs=[
            pl.BlockSpec((_EB, 128), lambda i: (i, 0)),
            pl.BlockSpec((2, _EB, 128), lambda i: (0, i, 0)),
            pl.BlockSpec((2, _EB, 128), lambda i: (0, i, 0)),
        ],
        out_shape=[
            jax.ShapeDtypeStruct((_EPAD, 128), F32),
            jax.ShapeDtypeStruct((2, _EPAD, 128), F32),
            jax.ShapeDtypeStruct((2, _EPAD, 128), F32),
        ],
    )(eap, We0, be0.reshape(1, -1), We1, be1.reshape(1, -1),
      We2, be2.reshape(1, -1))
    return t0, t1.reshape(2 * _EPAD, 128), t2.reshape(2 * _EPAD, 128)


def _mlp(h, a, cc, W1, b1, W2, b2, *, first, last_relu):
    grid = _N // _NB

    def body(h_ref, a_ref, c_ref, W1_ref, b1_ref, W2_ref, b2_ref, o_ref):
        if first:
            z = h_ref[...] + a_ref[0] + a_ref[1]
        else:
            z = jnp.concatenate([h_ref[0] + a_ref[0],
                                 h_ref[1] + a_ref[1]], axis=1)
        z = jnp.concatenate([z, c_ref[...]], axis=1)
        hmid = jnp.maximum(
            jnp.dot(z, W1_ref[...], preferred_element_type=F32) + b1_ref[...],
            0.0)
        hnew = jnp.dot(hmid, W2_ref[...], preferred_element_type=F32) \
            + b2_ref[...]
        if last_relu:
            hnew = jnp.maximum(hnew, 0.0)
        o_ref[0] = hnew[:, :128]
        o_ref[1] = hnew[:, 128:]

    din = 128 if first else 256
    h_spec = (pl.BlockSpec((_NB, 128), lambda i: (i, 0)) if first
              else pl.BlockSpec((2, _NB, 128), lambda i: (0, i, 0)))
    out = pl.pallas_call(
        body,
        grid=(grid,),
        in_specs=[
            h_spec,
            pl.BlockSpec((2, _NB, 128), lambda i: (0, i, 0)),
            pl.BlockSpec((_NB, 16), lambda i: (i, 0)),
            pl.BlockSpec((din + 16, 512), lambda i: (0, 0)),
            pl.BlockSpec((1, 512), lambda i: (0, 0)),
            pl.BlockSpec((512, 256), lambda i: (0, 0)),
            pl.BlockSpec((1, 256), lambda i: (0, 0)),
        ],
        out_specs=pl.BlockSpec((2, _NB, 128), lambda i: (0, i, 0)),
        out_shape=jax.ShapeDtypeStruct((2, _N2, 128), F32),
    )(h, a, cc, W1, b1.reshape(1, -1), W2, b2.reshape(1, -1))
    return out


def _readout(x, h1, h2, h3, batch3, Wp, bp, n_graphs):
    grid = _N // _NB

    def body(x_ref, h1_ref, h2_ref, h3_ref, b_ref, Wp_ref, bp_ref, o_ref,
             s_ref):
        i = pl.program_id(0)

        @pl.when(i == 0)
        def _():
            s_ref[...] = jnp.zeros_like(s_ref)

        b = b_ref[0, 0, :]
        iota = lax.broadcasted_iota(jnp.int32, (n_graphs, _NB), 0)
        onehot = (b[None, :] == iota).astype(F32)
        hcat = jnp.concatenate(
            [x_ref[...], h1_ref[0], h1_ref[1],
             h2_ref[0], h2_ref[1], h3_ref[0], h3_ref[1]], axis=1)
        s_ref[...] += jnp.dot(onehot, hcat, preferred_element_type=F32)

        @pl.when(i == grid - 1)
        def _():
            o_ref[...] = jnp.dot(s_ref[...], Wp_ref[...],
                                 preferred_element_type=F32) + bp_ref[...]

    return pl.pallas_call(
        body,
        grid=(grid,),
        in_specs=[
            pl.BlockSpec((_NB, 128), lambda i: (i, 0)),
            pl.BlockSpec((2, _NB, 128), lambda i: (0, i, 0)),
            pl.BlockSpec((2, _NB, 128), lambda i: (0, i, 0)),
            pl.BlockSpec((2, _NB, 128), lambda i: (0, i, 0)),
            pl.BlockSpec((1, 1, _NB), lambda i: (i, 0, 0)),
            pl.BlockSpec((896, 10), lambda i: (0, 0)),
            pl.BlockSpec((1, 10), lambda i: (0, 0)),
        ],
        out_specs=pl.BlockSpec((n_graphs, 10), lambda i: (0, 0)),
        out_shape=jax.ShapeDtypeStruct((n_graphs, 10), F32),
        scratch_shapes=[pltpu.VMEM((n_graphs, 896), F32)],
    )(x, h1, h2, h3, batch3, Wp, bp.reshape(1, -1))


def kernel(x, edge_index, c, edge_attr, batch,
           We0, be0, W10, b10, W20, b20,
           We1, be1, W11, b11, W21, b21,
           We2, be2, W12, b12, W22, b22,
           Wp, bp):
    pad = _EPAD - _E
    src = jnp.concatenate([edge_index[0], jnp.zeros((pad,), jnp.int32)])
    dst = jnp.concatenate([edge_index[1], jnp.full((pad,), _N, jnp.int32)])
    eap = jnp.concatenate([edge_attr, jnp.zeros((pad, 16), F32)])
    j_e = _mk_chunks(src, dst, e_pad=_EPAD, edge_split=True)
    j_c = _mk_chunks(src, dst, e_pad=_EPAD, edge_split=False)

    t0, t1, t2 = _edge_transform(eap, We0, be0, We1, be1, We2, be2)

    a0 = _edge_stage(x, j_e, t0, dh=128, e_pad=_EPAD, edge_split=True)
    h1 = _mlp(x, a0.reshape(2, _N2, 128), c, W10, b10, W20, b20,
              first=True, last_relu=True)

    a1 = _edge_stage(h1.reshape(2 * _N2, 128), j_c, t1,
                     dh=128, e_pad=_EPAD, edge_split=False)
    h2 = _mlp(h1, a1.reshape(2, _N2, 128), c, W11, b11, W21, b21,
              first=False, last_relu=True)

    a2 = _edge_stage(h2.reshape(2 * _N2, 128), j_c, t2,
                     dh=128, e_pad=_EPAD, edge_split=False)
    h3 = _mlp(h2, a2.reshape(2, _N2, 128), c, W12, b12, W22, b22,
              first=False, last_relu=False)

    batch3 = batch.reshape(_N // _NB, 1, _NB)
    return _readout(x, h1, h2, h3, batch3, Wp, bp, 64)
